# Initial kernel scaffold; baseline (speedup 1.0000x reference)
import jax
import jax.numpy as jnp
from jax.experimental import pallas as pl


def kernel(x, edge_index, att_weight):
    raise NotImplementedError("write your pallas kernel here")


# speedup vs baseline: 45.7069x; 45.7069x over previous
import functools

import jax
import jax.numpy as jnp
from jax import lax
from jax.experimental import pallas as pl
from jax.experimental.pallas import tpu as pltpu
from jax.experimental.pallas import tpu_sc as plsc

N_NODES = 10000
N_FEAT = 128
N_EDGES = 320000

NC = 2
NS = 16
NW = NC * NS
LANES = 16
CHUNK = N_EDGES // NW
STEPS = CHUNK // LANES

CLAMP_MIN = 1e-05
CLAMP_MAX = 0.99999


def _tc_scores_body(x_ref, w_ref, out_ref):
    out_ref[...] = lax.dot_general(
        w_ref[...],
        x_ref[...],
        (((1,), (1,)), ((), ())),
        preferred_element_type=jnp.float32,
        precision=lax.Precision.HIGHEST,
    )


def _sc_edge_body(edge_ref, s_ref, out_ref, src_v, dst_v, out_v, t_src, t_dst):
    wid = lax.axis_index("s") * NC + lax.axis_index("c")
    base = wid * CHUNK
    pltpu.sync_copy(s_ref.at[pl.ds(0, N_NODES)], t_src)
    pltpu.sync_copy(s_ref.at[pl.ds(N_NODES, N_NODES)], t_dst)
    pltpu.sync_copy(edge_ref.at[pl.ds(base, CHUNK)], src_v)
    pltpu.sync_copy(edge_ref.at[pl.ds(N_EDGES + base, CHUNK)], dst_v)

    @plsc.parallel_loop(0, CHUNK, LANES, unroll=8)
    def _step(i):
        sl = pl.ds(i, LANES)
        a = plsc.load_gather(t_src, [src_v[sl]]) + plsc.load_gather(
            t_dst, [dst_v[sl]]
        )
        sig = 1.0 / (1.0 + jnp.exp(-a))
        out_v[sl] = jnp.clip(sig, CLAMP_MIN, CLAMP_MAX)
    pltpu.sync_copy(out_v, out_ref.at[pl.ds(base, CHUNK)])


@jax.jit
def kernel(x, edge_index, att_weight):
    w = att_weight.reshape(2, N_FEAT)
    scores = pl.pallas_call(
        _tc_scores_body,
        out_shape=jax.ShapeDtypeStruct((2, N_NODES), jnp.float32),
    )(x, w)

    edges = edge_index.astype(jnp.int32).reshape(2 * N_EDGES)
    scores = scores.reshape(2 * N_NODES)
    sc_call = pl.kernel(
        _sc_edge_body,
        out_type=jax.ShapeDtypeStruct((N_EDGES,), jnp.float32),
        mesh=plsc.VectorSubcoreMesh(
            core_axis_name="c", subcore_axis_name="s", num_cores=NC, num_subcores=NS
        ),
        compiler_params=pltpu.CompilerParams(needs_layout_passes=False),
        scratch_types=[
            pltpu.VMEM((CHUNK,), jnp.int32),
            pltpu.VMEM((CHUNK,), jnp.int32),
            pltpu.VMEM((CHUNK,), jnp.float32),
            pltpu.VMEM((N_NODES,), jnp.float32),
            pltpu.VMEM((N_NODES,), jnp.float32),
        ],
    )
    return sc_call(edges, scores)


# speedup vs baseline: 48.7724x; 1.0671x over previous
import functools

import jax
import jax.numpy as jnp
from jax import lax
from jax.experimental import pallas as pl
from jax.experimental.pallas import tpu as pltpu
from jax.experimental.pallas import tpu_sc as plsc

N_NODES = 10000
N_FEAT = 128
N_EDGES = 320000

NC = 2
NS = 16
NW = NC * NS
LANES = 16
CHUNK = N_EDGES // NW
STEPS = CHUNK // LANES

CLAMP_MIN = 1e-05
CLAMP_MAX = 0.99999


ROW_BLOCK = 2048
N_ROW_BLOCKS = -(-N_NODES // ROW_BLOCK)


def _tc_scores_body(x_ref, w_ref, out_ref):
    out_ref[...] = lax.dot_general(
        w_ref[...],
        x_ref[...],
        (((1,), (1,)), ((), ())),
        preferred_element_type=jnp.float32,
        precision=lax.Precision.HIGHEST,
    )


def _sc_edge_body(
    edge_ref, s_ref, out_ref, src_v, dst_v, out_v, t_src, t_dst, sem
):
    wid = lax.axis_index("s") * NC + lax.axis_index("c")
    base = wid * CHUNK
    c1 = pltpu.async_copy(s_ref.at[pl.ds(0, N_NODES)], t_src, sem)
    c2 = pltpu.async_copy(s_ref.at[pl.ds(N_NODES, N_NODES)], t_dst, sem)
    c3 = pltpu.async_copy(edge_ref.at[pl.ds(base, CHUNK)], src_v, sem)
    c4 = pltpu.async_copy(edge_ref.at[pl.ds(N_EDGES + base, CHUNK)], dst_v, sem)
    c1.wait()
    c2.wait()
    c3.wait()
    c4.wait()

    @plsc.parallel_loop(0, CHUNK, LANES, unroll=8)
    def _step(i):
        sl = pl.ds(i, LANES)
        a = plsc.load_gather(t_src, [src_v[sl]]) + plsc.load_gather(
            t_dst, [dst_v[sl]]
        )
        sig = 1.0 / (1.0 + jnp.exp(-a))
        out_v[sl] = jnp.clip(sig, CLAMP_MIN, CLAMP_MAX)
    pltpu.sync_copy(out_v, out_ref.at[pl.ds(base, CHUNK)])


@jax.jit
def kernel(x, edge_index, att_weight):
    w = att_weight.reshape(2, N_FEAT)
    scores = pl.pallas_call(
        _tc_scores_body,
        grid=(N_ROW_BLOCKS,),
        in_specs=[
            pl.BlockSpec((ROW_BLOCK, N_FEAT), lambda i: (i, 0)),
            pl.BlockSpec((2, N_FEAT), lambda i: (0, 0)),
        ],
        out_specs=pl.BlockSpec((2, ROW_BLOCK), lambda i: (0, i)),
        out_shape=jax.ShapeDtypeStruct((2, N_NODES), jnp.float32),
    )(x, w)

    edges = edge_index.astype(jnp.int32).reshape(2 * N_EDGES)
    scores = scores.reshape(2 * N_NODES)
    sc_call = pl.kernel(
        _sc_edge_body,
        out_type=jax.ShapeDtypeStruct((N_EDGES,), jnp.float32),
        mesh=plsc.VectorSubcoreMesh(
            core_axis_name="c", subcore_axis_name="s", num_cores=NC, num_subcores=NS
        ),
        compiler_params=pltpu.CompilerParams(needs_layout_passes=False),
        scratch_types=[
            pltpu.VMEM((CHUNK,), jnp.int32),
            pltpu.VMEM((CHUNK,), jnp.int32),
            pltpu.VMEM((CHUNK,), jnp.float32),
            pltpu.VMEM((N_NODES,), jnp.float32),
            pltpu.VMEM((N_NODES,), jnp.float32),
            pltpu.SemaphoreType.DMA,
        ],
    )
    return sc_call(edges, scores)


# speedup vs baseline: 48.8104x; 1.0008x over previous
import functools

import jax
import jax.numpy as jnp
from jax import lax
from jax.experimental import pallas as pl
from jax.experimental.pallas import tpu as pltpu
from jax.experimental.pallas import tpu_sc as plsc

N_NODES = 10000
N_FEAT = 128
N_EDGES = 320000

NC = 2
NS = 16
NW = NC * NS
LANES = 16

SLAB = 10240
HALF = SLAB // 2
LAST_BASE = N_EDGES - SLAB
TAIL = N_EDGES - 31 * SLAB

CLAMP_MIN = 1e-05
CLAMP_MAX = 0.99999

ROW_BLOCK = 1024
N_ROW_BLOCKS = -(-N_NODES // ROW_BLOCK)


def _tc_scores_body(x_ref, w_ref, s_src_ref, s_dst_ref):
    xb = x_ref[...]
    s_src_ref[...] = jnp.sum(xb * w_ref[0:1, :], axis=1)
    s_dst_ref[...] = jnp.sum(xb * w_ref[1:2, :], axis=1)


def _sc_edge_body(
    edge_ref, s1_ref, s2_ref, out_ref, ev, out_v, t_src, t_dst, sem
):
    wid = lax.axis_index("s") * NC + lax.axis_index("c")
    is_last = wid == NW - 1
    base = jnp.where(is_last, LAST_BASE, wid * SLAB)

    c1 = pltpu.async_copy(s1_ref, t_src, sem)
    c2 = pltpu.async_copy(s2_ref, t_dst, sem)
    c3 = pltpu.async_copy(edge_ref.at[:, pl.ds(base, HALF)], ev.at[:, : HALF], sem)
    c4 = pltpu.async_copy(
        edge_ref.at[:, pl.ds(base + HALF, HALF)], ev.at[:, HALF:], sem
    )
    c1.wait()
    c2.wait()
    c3.wait()

    def make_step(lo):
        @plsc.parallel_loop(lo, lo + HALF, LANES, unroll=8)
        def _step(i):
            a = plsc.load_gather(t_src, [ev[0, pl.ds(i, LANES)]]) + plsc.load_gather(
                t_dst, [ev[1, pl.ds(i, LANES)]]
            )
            sig = 1.0 / (1.0 + jnp.exp(-a))
            out_v[pl.ds(i, LANES)] = jnp.clip(sig, CLAMP_MIN, CLAMP_MAX)

    make_step(0)
    c4.wait()
    make_step(HALF)

    @pl.when(jnp.logical_not(is_last))
    def _():
        pltpu.sync_copy(out_v, out_ref.at[pl.ds(wid * SLAB, SLAB)])

    @pl.when(is_last)
    def _():
        pltpu.sync_copy(
            out_v.at[pl.ds(SLAB - TAIL, TAIL)],
            out_ref.at[pl.ds(N_EDGES - TAIL, TAIL)],
        )


@jax.jit
def kernel(x, edge_index, att_weight):
    w = att_weight.reshape(2, N_FEAT)
    s_src, s_dst = pl.pallas_call(
        _tc_scores_body,
        grid=(N_ROW_BLOCKS,),
        in_specs=[
            pl.BlockSpec((ROW_BLOCK, N_FEAT), lambda i: (i, 0)),
            pl.BlockSpec((2, N_FEAT), lambda i: (0, 0)),
        ],
        out_specs=[
            pl.BlockSpec((ROW_BLOCK,), lambda i: (i,)),
            pl.BlockSpec((ROW_BLOCK,), lambda i: (i,)),
        ],
        out_shape=[
            jax.ShapeDtypeStruct((N_NODES,), jnp.float32),
            jax.ShapeDtypeStruct((N_NODES,), jnp.float32),
        ],
    )(x, w)

    edges = edge_index.astype(jnp.int32)
    sc_call = pl.kernel(
        _sc_edge_body,
        out_type=jax.ShapeDtypeStruct((N_EDGES,), jnp.float32),
        mesh=plsc.VectorSubcoreMesh(
            core_axis_name="c", subcore_axis_name="s", num_cores=NC, num_subcores=NS
        ),
        compiler_params=pltpu.CompilerParams(needs_layout_passes=False),
        scratch_types=[
            pltpu.VMEM((2, SLAB), jnp.int32),
            pltpu.VMEM((SLAB,), jnp.float32),
            pltpu.VMEM((N_NODES,), jnp.float32),
            pltpu.VMEM((N_NODES,), jnp.float32),
            pltpu.SemaphoreType.DMA,
        ],
    )
    return sc_call(edges, s_src, s_dst)


# speedup vs baseline: 55.3643x; 1.1343x over previous
import functools

import jax
import jax.numpy as jnp
from jax import lax
from jax.experimental import pallas as pl
from jax.experimental.pallas import tpu as pltpu
from jax.experimental.pallas import tpu_sc as plsc

N_NODES = 10000
N_FEAT = 128
N_EDGES = 320000

NC = 2
NS = 16
NW = NC * NS
LANES = 16

SLAB = 10240
N_PIECES = 4
PIECE = SLAB // N_PIECES
LAST_BASE = N_EDGES - SLAB
TAIL = N_EDGES - 31 * SLAB

CLAMP_MIN = 1e-05
CLAMP_MAX = 0.99999

ROW_BLOCK = 2048
N_ROW_BLOCKS = -(-N_NODES // ROW_BLOCK)


def _tc_scores_body(x_ref, w_ref, s_src_ref, s_dst_ref):
    res = lax.dot_general(
        w_ref[...],
        x_ref[...],
        (((1,), (1,)), ((), ())),
        preferred_element_type=jnp.float32,
        precision=lax.Precision.HIGHEST,
    )
    s_src_ref[...] = res[0]
    s_dst_ref[...] = res[1]


def _sc_edge_body(
    edge_ref, s1_ref, s2_ref, out_ref, ev, out_v, t_src, t_dst, sem
):
    wid = lax.axis_index("s") * NC + lax.axis_index("c")
    is_last = wid == NW - 1
    base = jnp.where(is_last, LAST_BASE, wid * SLAB)

    c1 = pltpu.async_copy(s1_ref, t_src, sem)
    c2 = pltpu.async_copy(s2_ref, t_dst, sem)
    pieces = [
        pltpu.async_copy(
            edge_ref.at[:, pl.ds(base + p * PIECE, PIECE)],
            ev.at[:, p * PIECE : (p + 1) * PIECE],
            sem,
        )
        for p in range(N_PIECES)
    ]
    c1.wait()
    c2.wait()

    def make_step(lo):
        @plsc.parallel_loop(lo, lo + PIECE, LANES, unroll=4)
        def _step(i):
            a = plsc.load_gather(t_src, [ev[0, pl.ds(i, LANES)]]) + plsc.load_gather(
                t_dst, [ev[1, pl.ds(i, LANES)]]
            )
            sig = 1.0 / (1.0 + jnp.exp(-a))
            out_v[pl.ds(i, LANES)] = jnp.clip(sig, CLAMP_MIN, CLAMP_MAX)

    for p in range(N_PIECES):
        pieces[p].wait()
        make_step(p * PIECE)

    @pl.when(jnp.logical_not(is_last))
    def _():
        pltpu.sync_copy(out_v, out_ref.at[pl.ds(wid * SLAB, SLAB)])

    @pl.when(is_last)
    def _():
        pltpu.sync_copy(
            out_v.at[pl.ds(SLAB - TAIL, TAIL)],
            out_ref.at[pl.ds(N_EDGES - TAIL, TAIL)],
        )


@jax.jit
def kernel(x, edge_index, att_weight):
    w = att_weight.reshape(2, N_FEAT)
    s_src, s_dst = pl.pallas_call(
        _tc_scores_body,
        grid=(N_ROW_BLOCKS,),
        in_specs=[
            pl.BlockSpec((ROW_BLOCK, N_FEAT), lambda i: (i, 0)),
            pl.BlockSpec((2, N_FEAT), lambda i: (0, 0)),
        ],
        out_specs=[
            pl.BlockSpec((ROW_BLOCK,), lambda i: (i,)),
            pl.BlockSpec((ROW_BLOCK,), lambda i: (i,)),
        ],
        out_shape=[
            jax.ShapeDtypeStruct((N_NODES,), jnp.float32),
            jax.ShapeDtypeStruct((N_NODES,), jnp.float32),
        ],
    )(x, w)

    edges = edge_index.astype(jnp.int32)
    sc_call = pl.kernel(
        _sc_edge_body,
        out_type=jax.ShapeDtypeStruct((N_EDGES,), jnp.float32),
        mesh=plsc.VectorSubcoreMesh(
            core_axis_name="c", subcore_axis_name="s", num_cores=NC, num_subcores=NS
        ),
        compiler_params=pltpu.CompilerParams(needs_layout_passes=False),
        scratch_types=[
            pltpu.VMEM((2, SLAB), jnp.int32),
            pltpu.VMEM((SLAB,), jnp.float32),
            pltpu.VMEM((N_NODES,), jnp.float32),
            pltpu.VMEM((N_NODES,), jnp.float32),
            pltpu.SemaphoreType.DMA,
        ],
    )
    return sc_call(edges, s_src, s_dst)


# speedup vs baseline: 61.4945x; 1.1107x over previous
import functools

import jax
import jax.numpy as jnp
from jax import lax
from jax.experimental import pallas as pl
from jax.experimental.pallas import tpu as pltpu
from jax.experimental.pallas import tpu_sc as plsc

N_NODES = 10000
N_FEAT = 128
N_EDGES = 320000

NC = 2
NS = 16
NW = NC * NS
LANES = 16

SLAB = 10240
N_PIECES = 4
PIECE = SLAB // N_PIECES
LAST_BASE = N_EDGES - SLAB
TAIL = N_EDGES - 31 * SLAB

CLAMP_MIN = 1e-05
CLAMP_MAX = 0.99999

ROW_BLOCK = 4096
N_ROW_BLOCKS = -(-N_NODES // ROW_BLOCK)


def _tc_scores_body(x_ref, w_ref, s_src_ref, s_dst_ref):
    res = lax.dot_general(
        w_ref[...],
        x_ref[...],
        (((1,), (1,)), ((), ())),
        preferred_element_type=jnp.float32,
        precision=lax.Precision.DEFAULT,
    )
    s_src_ref[...] = res[0]
    s_dst_ref[...] = res[1]


def _sc_edge_body(
    edge_ref, s1_ref, s2_ref, out_ref, ev, out_v, t_src, t_dst, sem
):
    wid = lax.axis_index("s") * NC + lax.axis_index("c")
    is_last = wid == NW - 1
    base = jnp.where(is_last, LAST_BASE, wid * SLAB)

    c1 = pltpu.async_copy(s1_ref, t_src, sem)
    c2 = pltpu.async_copy(s2_ref, t_dst, sem)
    pieces = [
        pltpu.async_copy(
            edge_ref.at[:, pl.ds(base + p * PIECE, PIECE)],
            ev.at[:, p * PIECE : (p + 1) * PIECE],
            sem,
        )
        for p in range(N_PIECES)
    ]
    c1.wait()
    c2.wait()

    def make_step(lo):
        @plsc.parallel_loop(lo, lo + PIECE, LANES, unroll=8)
        def _step(i):
            a = plsc.load_gather(t_src, [ev[0, pl.ds(i, LANES)]]) + plsc.load_gather(
                t_dst, [ev[1, pl.ds(i, LANES)]]
            )
            sig = 1.0 / (1.0 + jnp.exp(-a))
            out_v[pl.ds(i, LANES)] = jnp.clip(sig, CLAMP_MIN, CLAMP_MAX)

    for p in range(N_PIECES):
        pieces[p].wait()
        make_step(p * PIECE)

    @pl.when(jnp.logical_not(is_last))
    def _():
        pltpu.sync_copy(out_v, out_ref.at[pl.ds(wid * SLAB, SLAB)])

    @pl.when(is_last)
    def _():
        pltpu.sync_copy(
            out_v.at[pl.ds(SLAB - TAIL, TAIL)],
            out_ref.at[pl.ds(N_EDGES - TAIL, TAIL)],
        )


@jax.jit
def kernel(x, edge_index, att_weight):
    w = att_weight.reshape(2, N_FEAT)
    s_src, s_dst = pl.pallas_call(
        _tc_scores_body,
        grid=(N_ROW_BLOCKS,),
        in_specs=[
            pl.BlockSpec((ROW_BLOCK, N_FEAT), lambda i: (i, 0)),
            pl.BlockSpec((2, N_FEAT), lambda i: (0, 0)),
        ],
        out_specs=[
            pl.BlockSpec((ROW_BLOCK,), lambda i: (i,)),
            pl.BlockSpec((ROW_BLOCK,), lambda i: (i,)),
        ],
        out_shape=[
            jax.ShapeDtypeStruct((N_NODES,), jnp.float32),
            jax.ShapeDtypeStruct((N_NODES,), jnp.float32),
        ],
    )(x, w)

    edges = edge_index.astype(jnp.int32)
    sc_call = pl.kernel(
        _sc_edge_body,
        out_type=jax.ShapeDtypeStruct((N_EDGES,), jnp.float32),
        mesh=plsc.VectorSubcoreMesh(
            core_axis_name="c", subcore_axis_name="s", num_cores=NC, num_subcores=NS
        ),
        compiler_params=pltpu.CompilerParams(needs_layout_passes=False),
        scratch_types=[
            pltpu.VMEM((2, SLAB), jnp.int32),
            pltpu.VMEM((SLAB,), jnp.float32),
            pltpu.VMEM((N_NODES,), jnp.float32),
            pltpu.VMEM((N_NODES,), jnp.float32),
            pltpu.SemaphoreType.DMA,
        ],
    )
    return sc_call(edges, s_src, s_dst)
